# Initial kernel scaffold; baseline (speedup 1.0000x reference)
import jax
import jax.numpy as jnp
from jax.experimental import pallas as pl


def kernel(k, ak, bk, gk, mk):
    raise NotImplementedError("write your pallas kernel here")


# speedup vs baseline: 134.0559x; 134.0559x over previous
import functools

import jax
import jax.numpy as jnp
from jax import lax
from jax.experimental import pallas as pl
from jax.experimental.pallas import tpu as pltpu
from jax.experimental.pallas import tpu_sc as plsc

_NUM_CORES = 2
_NUM_SUBCORES = 16
_NUM_WORKERS = _NUM_CORES * _NUM_SUBCORES

_D = 8
_GRP = 128
_LANES = 16


def _gather_rows(kf, table, *, window):
    n = kf.shape[0]
    per_w = n // _NUM_WORKERS
    n_win = per_w // window
    n_grp = window // _GRP
    assert per_w % window == 0 and window % _GRP == 0

    mesh = plsc.VectorSubcoreMesh(
        core_axis_name="c", subcore_axis_name="s",
        num_cores=_NUM_CORES, num_subcores=_NUM_SUBCORES)

    @functools.partial(
        pl.kernel,
        out_type=jax.ShapeDtypeStruct((4, n), jnp.float32),
        mesh=mesh,
        scratch_types=[
            pltpu.VMEM((window,), jnp.int32),
            pltpu.VMEM((window, _D), jnp.float32),
            pltpu.VMEM((4, window), jnp.float32),
            pltpu.SemaphoreType.DMA,
        ],
        compiler_params=pltpu.CompilerParams(
            use_tc_tiling_on_sc=False, needs_layout_passes=False),
    )
    def body(k_hbm, tab_hbm, out_hbm, idx_v, buf_v, cols_v, sem):
        wid = lax.axis_index("s") * _NUM_CORES + lax.axis_index("c")
        base = wid * per_w
        lanes = lax.iota(jnp.int32, _LANES)

        def step(w, carry):
            off = base + w * window
            pltpu.sync_copy(k_hbm.at[pl.ds(off, window)], idx_v)
            copies = [
                pltpu.async_copy(
                    tab_hbm.at[idx_v.at[pl.ds(j * _GRP, _GRP)]],
                    buf_v.at[pl.ds(j * _GRP, _GRP)], sem)
                for j in range(n_grp)
            ]
            for cp in copies:
                cp.wait()
            for v in range(window // _LANES):
                rows = lanes + (v * _LANES)
                for t in range(4):
                    vec = plsc.load_gather(
                        buf_v, [rows, jnp.full((_LANES,), t, jnp.int32)])
                    cols_v[t, pl.ds(v * _LANES, _LANES)] = vec
            for t in range(4):
                pltpu.sync_copy(cols_v.at[t], out_hbm.at[t, pl.ds(off, window)])
            return carry

        lax.fori_loop(0, n_win, step, 0)

    return body(kf, table)


def kernel(k, ak, bk, gk, mk):
    b, l = k.shape
    n = b * l
    kf = k.reshape(n).astype(jnp.int32)
    zero = jnp.zeros_like(ak)
    table = jnp.stack([ak, bk, gk, mk, zero, zero, zero, zero], axis=1)
    out = _gather_rows(kf, table, window=2048)
    return out.reshape(4, b, l)


# speedup vs baseline: 147.5133x; 1.1004x over previous
import functools

import jax
import jax.numpy as jnp
from jax import lax
from jax.experimental import pallas as pl
from jax.experimental.pallas import tpu as pltpu
from jax.experimental.pallas import tpu_sc as plsc

_NUM_CORES = 2
_NUM_SUBCORES = 16
_NUM_WORKERS = _NUM_CORES * _NUM_SUBCORES

_D = 8
_GRP = 128
_LANES = 16


def _gather_rows(kf, table, *, window):
    n = kf.shape[0]
    per_w = n // _NUM_WORKERS
    n_win = per_w // window
    n_grp = window // _GRP
    assert per_w % window == 0 and window % _GRP == 0 and n_win % 2 == 0

    mesh = plsc.VectorSubcoreMesh(
        core_axis_name="c", subcore_axis_name="s",
        num_cores=_NUM_CORES, num_subcores=_NUM_SUBCORES)

    @functools.partial(
        pl.kernel,
        out_type=jax.ShapeDtypeStruct((4, n), jnp.float32),
        mesh=mesh,
        scratch_types=[
            pltpu.VMEM((window,), jnp.int32),
            pltpu.VMEM((window,), jnp.int32),
            pltpu.VMEM((window, _D), jnp.float32),
            pltpu.VMEM((window, _D), jnp.float32),
            pltpu.VMEM((4, window), jnp.float32),
            pltpu.SemaphoreType.DMA,
            pltpu.SemaphoreType.DMA,
        ],
        compiler_params=pltpu.CompilerParams(
            use_tc_tiling_on_sc=False, needs_layout_passes=False),
    )
    def body(k_hbm, tab_hbm, out_hbm, idx0, idx1, buf0, buf1, cols_v,
             sem0, sem1):
        wid = lax.axis_index("s") * _NUM_CORES + lax.axis_index("c")
        base = wid * per_w
        lanes = lax.iota(jnp.int32, _LANES)
        idx_b = (idx0, idx1)
        buf_b = (buf0, buf1)
        sem_b = (sem0, sem1)

        def launch(w, p):
            off = base + w * window
            pltpu.sync_copy(k_hbm.at[pl.ds(off, window)], idx_b[p])
            for j in range(n_grp):
                pltpu.async_copy(
                    tab_hbm.at[idx_b[p].at[pl.ds(j * _GRP, _GRP)]],
                    buf_b[p].at[pl.ds(j * _GRP, _GRP)], sem_b[p])

        def finish(w, p):
            off = base + w * window
            for j in range(n_grp):
                pltpu.make_async_copy(
                    tab_hbm.at[idx_b[p].at[pl.ds(j * _GRP, _GRP)]],
                    buf_b[p].at[pl.ds(j * _GRP, _GRP)], sem_b[p]).wait()
            for v in range(window // _LANES):
                rows = lanes + (v * _LANES)
                for t in range(4):
                    vec = plsc.load_gather(
                        buf_b[p], [rows, jnp.full((_LANES,), t, jnp.int32)])
                    cols_v[t, pl.ds(v * _LANES, _LANES)] = vec
            for t in range(4):
                pltpu.sync_copy(cols_v.at[t], out_hbm.at[t, pl.ds(off, window)])

        launch(0, 0)

        def step(i, carry):
            w0 = 2 * i
            launch(w0 + 1, 1)
            finish(w0, 0)

            @pl.when(i < n_win // 2 - 1)
            def _():
                launch(w0 + 2, 0)

            finish(w0 + 1, 1)
            return carry

        lax.fori_loop(0, n_win // 2, step, 0)

    return body(kf, table)


def kernel(k, ak, bk, gk, mk):
    b, l = k.shape
    n = b * l
    kf = k.reshape(n).astype(jnp.int32)
    zero = jnp.zeros_like(ak)
    table = jnp.stack([ak, bk, gk, mk, zero, zero, zero, zero], axis=1)
    out = _gather_rows(kf, table, window=2048)
    return out.reshape(4, b, l)


# speedup vs baseline: 155.0049x; 1.0508x over previous
import functools

import jax
import jax.numpy as jnp
from jax import lax
from jax.experimental import pallas as pl
from jax.experimental.pallas import tpu as pltpu
from jax.experimental.pallas import tpu_sc as plsc

_NUM_CORES = 2
_NUM_SUBCORES = 16
_NUM_WORKERS = _NUM_CORES * _NUM_SUBCORES

_D = 8
_GRP = 128
_LANES = 16


def _gather_rows(kf, table, *, window):
    n = kf.shape[0]
    per_w = n // _NUM_WORKERS
    n_win = per_w // window
    n_grp = window // _GRP
    assert per_w % window == 0 and window % _GRP == 0 and n_win % 2 == 0

    mesh = plsc.VectorSubcoreMesh(
        core_axis_name="c", subcore_axis_name="s",
        num_cores=_NUM_CORES, num_subcores=_NUM_SUBCORES)

    @functools.partial(
        pl.kernel,
        out_type=jax.ShapeDtypeStruct((4, n), jnp.float32),
        mesh=mesh,
        scratch_types=[
            pltpu.VMEM((window,), jnp.int32),
            pltpu.VMEM((window,), jnp.int32),
            pltpu.VMEM((window, _D), jnp.float32),
            pltpu.VMEM((window, _D), jnp.float32),
            pltpu.VMEM((4, window), jnp.float32),
            pltpu.SemaphoreType.DMA,
            pltpu.SemaphoreType.DMA,
        ],
        compiler_params=pltpu.CompilerParams(
            use_tc_tiling_on_sc=False, needs_layout_passes=False),
    )
    def body(k_hbm, tab_hbm, out_hbm, idx0, idx1, buf0, buf1, cols_v,
             sem0, sem1):
        wid = lax.axis_index("s") * _NUM_CORES + lax.axis_index("c")
        base = wid * per_w
        lanes = lax.iota(jnp.int32, _LANES)
        idx_b = (idx0, idx1)
        buf_b = (buf0, buf1)
        sem_b = (sem0, sem1)

        def launch(w, p):
            off = base + w * window
            pltpu.sync_copy(k_hbm.at[pl.ds(off, window)], idx_b[p])
            for j in range(n_grp):
                pltpu.async_copy(
                    tab_hbm.at[idx_b[p].at[pl.ds(j * _GRP, _GRP)]],
                    buf_b[p].at[pl.ds(j * _GRP, _GRP)], sem_b[p])

        def finish(w, p):
            off = base + w * window
            for j in range(n_grp):
                pltpu.make_async_copy(
                    tab_hbm.at[idx_b[p].at[pl.ds(j * _GRP, _GRP)]],
                    buf_b[p].at[pl.ds(j * _GRP, _GRP)], sem_b[p]).wait()
            def deint(v, c2):
                o = v * _LANES
                rows = lanes + o
                for t in range(4):
                    vec = plsc.load_gather(
                        buf_b[p], [rows, jnp.full((_LANES,), t, jnp.int32)])
                    cols_v[t, pl.ds(o, _LANES)] = vec
                return c2

            lax.fori_loop(0, window // _LANES, deint, 0, unroll=4)
            for t in range(4):
                pltpu.sync_copy(cols_v.at[t], out_hbm.at[t, pl.ds(off, window)])

        launch(0, 0)

        def step(i, carry):
            w0 = 2 * i
            launch(w0 + 1, 1)
            finish(w0, 0)

            @pl.when(i < n_win // 2 - 1)
            def _():
                launch(w0 + 2, 0)

            finish(w0 + 1, 1)
            return carry

        lax.fori_loop(0, n_win // 2, step, 0)

    return body(kf, table)


def kernel(k, ak, bk, gk, mk):
    b, l = k.shape
    n = b * l
    kf = k.reshape(n).astype(jnp.int32)
    zero = jnp.zeros_like(ak)
    table = jnp.stack([ak, bk, gk, mk, zero, zero, zero, zero], axis=1)
    out = _gather_rows(kf, table, window=2048)
    return out.reshape(4, b, l)


# speedup vs baseline: 263.6601x; 1.7010x over previous
import functools

import jax
import jax.numpy as jnp
from jax import lax
from jax.experimental import pallas as pl
from jax.experimental.pallas import tpu as pltpu
from jax.experimental.pallas import tpu_sc as plsc

_NUM_CORES = 2
_NUM_SUBCORES = 16
_NUM_WORKERS = _NUM_CORES * _NUM_SUBCORES

_D = 8
_GRP = 128
_LANES = 16
_IW = 2000


def _gather4(kf, ak, bk, gk, mk, *, window):
    n = kf.shape[0]
    v_rows = ak.shape[0]
    per_w = n // _NUM_WORKERS
    n_win = per_w // window
    n_grp = window // _GRP
    assert per_w % window == 0 and window % _GRP == 0 and n_win % 2 == 0
    n_iwin = v_rows // _IW
    assert v_rows % _IW == 0 and _IW % _LANES == 0
    iwin_rem = n_iwin % _NUM_SUBCORES

    mesh = plsc.VectorSubcoreMesh(
        core_axis_name="c", subcore_axis_name="s",
        num_cores=_NUM_CORES, num_subcores=_NUM_SUBCORES)

    @functools.partial(
        pl.kernel,
        out_type=(
            jax.ShapeDtypeStruct((4, n), jnp.float32),
            jax.ShapeDtypeStruct((_NUM_CORES, v_rows, _D), jnp.float32),
        ),
        mesh=mesh,
        scratch_types=[
            pltpu.VMEM((window,), jnp.int32),
            pltpu.VMEM((window,), jnp.int32),
            pltpu.VMEM((window, _D), jnp.float32),
            pltpu.VMEM((window, _D), jnp.float32),
            pltpu.VMEM((4, window), jnp.float32),
            pltpu.VMEM((4, _IW), jnp.float32),
            pltpu.VMEM((_IW, _D), jnp.float32),
            pltpu.SemaphoreType.DMA,
            pltpu.SemaphoreType.DMA,
        ],
        compiler_params=pltpu.CompilerParams(
            use_tc_tiling_on_sc=False, needs_layout_passes=False),
    )
    def body(k_hbm, a_hbm, b_hbm, g_hbm, m_hbm, out_hbm, tab_hbm,
             idx0, idx1, buf0, buf1, cols_v, stg_v, ibuf_v, sem0, sem1):
        cid = lax.axis_index("c")
        sid = lax.axis_index("s")
        wid = sid * _NUM_CORES + cid
        base = wid * per_w
        lanes = lax.iota(jnp.int32, _LANES)
        idx_b = (idx0, idx1)
        buf_b = (buf0, buf1)
        sem_b = (sem0, sem1)

        tcols = [jnp.full((_LANES,), t, jnp.int32) for t in range(4)]

        def iw_step(i, carry):
            win = i * _NUM_SUBCORES + sid
            r0 = win * _IW
            for t, src in enumerate((a_hbm, b_hbm, g_hbm, m_hbm)):
                pltpu.sync_copy(src.at[pl.ds(r0, _IW)], stg_v.at[t])

            def ivec(v, c2):
                o = v * _LANES
                rows = lanes + o
                for t in range(4):
                    plsc.store_scatter(ibuf_v, [rows, tcols[t]],
                                       stg_v[t, pl.ds(o, _LANES)])
                return c2

            lax.fori_loop(0, _IW // _LANES, ivec, 0, unroll=4)
            pltpu.sync_copy(ibuf_v, tab_hbm.at[cid, pl.ds(r0, _IW)])
            return carry

        n_my = jnp.where(sid < iwin_rem, n_iwin // _NUM_SUBCORES + 1,
                         n_iwin // _NUM_SUBCORES)
        lax.fori_loop(0, n_my, iw_step, 0)
        plsc.subcore_barrier()

        def launch(w, p):
            off = base + w * window
            pltpu.sync_copy(k_hbm.at[pl.ds(off, window)], idx_b[p])
            for j in range(n_grp):
                pltpu.async_copy(
                    tab_hbm.at[cid].at[idx_b[p].at[pl.ds(j * _GRP, _GRP)]],
                    buf_b[p].at[pl.ds(j * _GRP, _GRP)], sem_b[p])

        def finish(w, p):
            off = base + w * window
            for j in range(n_grp):
                pltpu.make_async_copy(
                    tab_hbm.at[cid].at[idx_b[p].at[pl.ds(j * _GRP, _GRP)]],
                    buf_b[p].at[pl.ds(j * _GRP, _GRP)], sem_b[p]).wait()

            def deint(v, c2):
                o = v * _LANES
                rows = lanes + o
                for t in range(4):
                    vec = plsc.load_gather(buf_b[p], [rows, tcols[t]])
                    cols_v[t, pl.ds(o, _LANES)] = vec
                return c2

            lax.fori_loop(0, window // _LANES, deint, 0, unroll=4)
            for t in range(4):
                pltpu.sync_copy(cols_v.at[t], out_hbm.at[t, pl.ds(off, window)])

        launch(0, 0)

        def step(i, carry):
            w0 = 2 * i
            launch(w0 + 1, 1)
            finish(w0, 0)

            @pl.when(i < n_win // 2 - 1)
            def _():
                launch(w0 + 2, 0)

            finish(w0 + 1, 1)
            return carry

        lax.fori_loop(0, n_win // 2, step, 0)

    out, _ = body(kf, ak, bk, gk, mk)
    return out


def kernel(k, ak, bk, gk, mk):
    b, l = k.shape
    n = b * l
    kf = k.reshape(n).astype(jnp.int32)
    out = _gather4(kf, ak, bk, gk, mk, window=2048)
    return out.reshape(4, b, l)


# speedup vs baseline: 292.9168x; 1.1110x over previous
import functools

import jax
import jax.numpy as jnp
from jax import lax
from jax.experimental import pallas as pl
from jax.experimental.pallas import tpu as pltpu
from jax.experimental.pallas import tpu_sc as plsc

_NUM_CORES = 2
_NUM_SUBCORES = 16
_NUM_WORKERS = _NUM_CORES * _NUM_SUBCORES

_D = 8
_GRP = 128
_LANES = 16
_IW = 2000
_WROWS = 16


def _gather4(kf, ak, bk, gk, mk, *, b, l):
    n = kf.shape[0]
    v_rows = ak.shape[0]
    rows_per_w = b // _NUM_WORKERS
    window = _WROWS * l
    per_w = rows_per_w * l
    n_win = rows_per_w // _WROWS
    n_grp = window // _GRP
    assert rows_per_w % _WROWS == 0 and window % _GRP == 0 and n_win % 2 == 0
    n_iwin = v_rows // _IW
    assert v_rows % _IW == 0 and _IW % _LANES == 0
    n_my = -(-n_iwin // _NUM_SUBCORES)
    chunk0 = list(range(0, l - _LANES + 1, _LANES))
    if chunk0[-1] != l - _LANES:
        chunk0.append(l - _LANES)

    mesh = plsc.VectorSubcoreMesh(
        core_axis_name="c", subcore_axis_name="s",
        num_cores=_NUM_CORES, num_subcores=_NUM_SUBCORES)

    @functools.partial(
        pl.kernel,
        out_type=(
            jax.ShapeDtypeStruct((4, b, l), jnp.float32),
            jax.ShapeDtypeStruct((_NUM_CORES, v_rows, _D), jnp.float32),
        ),
        mesh=mesh,
        scratch_types=[
            pltpu.VMEM((window,), jnp.int32),
            pltpu.VMEM((window,), jnp.int32),
            pltpu.VMEM((window, _D), jnp.float32),
            pltpu.VMEM((window, _D), jnp.float32),
            pltpu.VMEM((4, _WROWS, l), jnp.float32),
            pltpu.VMEM((4, _IW), jnp.float32),
            pltpu.VMEM((_IW, _D), jnp.float32),
            pltpu.SemaphoreType.DMA,
            pltpu.SemaphoreType.DMA,
            pltpu.SemaphoreType.DMA,
        ],
        compiler_params=pltpu.CompilerParams(
            use_tc_tiling_on_sc=False, needs_layout_passes=False),
    )
    def body(k_hbm, a_hbm, b_hbm, g_hbm, m_hbm, out_hbm, tab_hbm,
             idx0, idx1, buf0, buf1, cols_v, stg_v, ibuf_v, sem0, sem1, semi):
        cid = lax.axis_index("c")
        sid = lax.axis_index("s")
        wid = sid * _NUM_CORES + cid
        base = wid * per_w
        row_base = wid * rows_per_w
        lanes = lax.iota(jnp.int32, _LANES)
        idx_b = (idx0, idx1)
        buf_b = (buf0, buf1)
        sem_b = (sem0, sem1)
        tcols = [jnp.full((_LANES,), t, jnp.int32) for t in range(4)]

        def iw_step(i, carry):
            win = jnp.minimum(i * _NUM_SUBCORES + sid, n_iwin - 1)
            r0 = win * _IW
            srcs = (a_hbm, b_hbm, g_hbm, m_hbm)
            for t, src in enumerate(srcs):
                pltpu.async_copy(src.at[pl.ds(r0, _IW)], stg_v.at[t], semi)
            for t, src in enumerate(srcs):
                pltpu.make_async_copy(
                    src.at[pl.ds(r0, _IW)], stg_v.at[t], semi).wait()

            def ivec(v, c2):
                o = v * _LANES
                rows = lanes + o
                for t in range(4):
                    plsc.store_scatter(ibuf_v, [rows, tcols[t]],
                                       stg_v[t, pl.ds(o, _LANES)])
                return c2

            lax.fori_loop(0, _IW // _LANES, ivec, 0, unroll=4)
            pltpu.sync_copy(ibuf_v, tab_hbm.at[cid, pl.ds(r0, _IW)])
            return carry

        lax.fori_loop(0, n_my, iw_step, 0)
        plsc.subcore_barrier()

        def launch(w, p):
            off = base + w * window
            pltpu.sync_copy(k_hbm.at[pl.ds(off, window)], idx_b[p])
            for j in range(n_grp):
                pltpu.async_copy(
                    tab_hbm.at[cid].at[idx_b[p].at[pl.ds(j * _GRP, _GRP)]],
                    buf_b[p].at[pl.ds(j * _GRP, _GRP)], sem_b[p])

        def finish(w, p):
            row0 = row_base + w * _WROWS
            for j in range(n_grp):
                pltpu.make_async_copy(
                    tab_hbm.at[cid].at[idx_b[p].at[pl.ds(j * _GRP, _GRP)]],
                    buf_b[p].at[pl.ds(j * _GRP, _GRP)], sem_b[p]).wait()

            def deint(r, c2):
                p0 = r * l
                for c0 in chunk0:
                    rows = lanes + (p0 + c0)
                    for t in range(4):
                        vec = plsc.load_gather(buf_b[p], [rows, tcols[t]])
                        cols_v[t, r, pl.ds(c0, _LANES)] = vec
                return c2

            lax.fori_loop(0, _WROWS, deint, 0)
            for t in range(4):
                pltpu.sync_copy(cols_v.at[t],
                                out_hbm.at[t, pl.ds(row0, _WROWS)])

        launch(0, 0)

        def step(i, carry):
            w0 = 2 * i
            launch(w0 + 1, 1)
            finish(w0, 0)

            @pl.when(i < n_win // 2 - 1)
            def _():
                launch(w0 + 2, 0)

            finish(w0 + 1, 1)
            return carry

        lax.fori_loop(0, n_win // 2, step, 0)

    out, _ = body(kf, ak, bk, gk, mk)
    return out


def kernel(k, ak, bk, gk, mk):
    b, l = k.shape
    kf = k.reshape(b * l).astype(jnp.int32)
    return _gather4(kf, ak, bk, gk, mk, b=b, l=l)


# speedup vs baseline: 315.3081x; 1.0764x over previous
import functools

import jax
import jax.numpy as jnp
from jax import lax
from jax.experimental import pallas as pl
from jax.experimental.pallas import tpu as pltpu
from jax.experimental.pallas import tpu_sc as plsc

_NUM_CORES = 2
_NUM_SUBCORES = 16
_NUM_WORKERS = _NUM_CORES * _NUM_SUBCORES

_D = 8
_GRP = 128
_LANES = 16
_IW = 2000
_WROWS = 16


def _gather4(kf, ak, bk, gk, mk, *, b, l):
    n = kf.shape[0]
    v_rows = ak.shape[0]
    rows_per_w = b // _NUM_WORKERS
    window = _WROWS * l
    per_w = rows_per_w * l
    n_win = rows_per_w // _WROWS
    n_grp = window // _GRP
    assert rows_per_w % _WROWS == 0 and window % _GRP == 0 and n_win % 2 == 0
    n_iwin = v_rows // _IW
    assert v_rows % _IW == 0 and _IW % _LANES == 0
    n_my = -(-n_iwin // _NUM_SUBCORES)
    assert n_my % 2 == 0
    chunk0 = list(range(0, l - _LANES + 1, _LANES))
    if chunk0[-1] != l - _LANES:
        chunk0.append(l - _LANES)

    mesh = plsc.VectorSubcoreMesh(
        core_axis_name="c", subcore_axis_name="s",
        num_cores=_NUM_CORES, num_subcores=_NUM_SUBCORES)

    @functools.partial(
        pl.kernel,
        out_type=(
            jax.ShapeDtypeStruct((4, b, l), jnp.float32),
            jax.ShapeDtypeStruct((_NUM_CORES, v_rows, _D), jnp.float32),
        ),
        mesh=mesh,
        scratch_types=[
            pltpu.VMEM((window,), jnp.int32),
            pltpu.VMEM((window,), jnp.int32),
            pltpu.VMEM((window, _D), jnp.float32),
            pltpu.VMEM((window, _D), jnp.float32),
            pltpu.VMEM((4, _WROWS, l), jnp.float32),
            pltpu.VMEM((4, _WROWS, l), jnp.float32),
            pltpu.VMEM((4, _IW), jnp.float32),
            pltpu.VMEM((_IW, _D), jnp.float32),
            pltpu.VMEM((_IW, _D), jnp.float32),
            pltpu.SemaphoreType.DMA,
            pltpu.SemaphoreType.DMA,
            pltpu.SemaphoreType.DMA,
            pltpu.SemaphoreType.DMA,
            pltpu.SemaphoreType.DMA,
            pltpu.SemaphoreType.DMA,
            pltpu.SemaphoreType.DMA,
        ],
        compiler_params=pltpu.CompilerParams(
            use_tc_tiling_on_sc=False, needs_layout_passes=False),
    )
    def body(k_hbm, a_hbm, b_hbm, g_hbm, m_hbm, out_hbm, tab_hbm,
             idx0, idx1, buf0, buf1, cols0, cols1, stg_v, ibuf0, ibuf1,
             sem0, sem1, semo0, semo1, semi, semt0, semt1):
        cid = lax.axis_index("c")
        sid = lax.axis_index("s")
        wid = sid * _NUM_CORES + cid
        base = wid * per_w
        row_base = wid * rows_per_w
        lanes = lax.iota(jnp.int32, _LANES)
        idx_b = (idx0, idx1)
        buf_b = (buf0, buf1)
        cols_b = (cols0, cols1)
        sem_b = (sem0, sem1)
        semo_b = (semo0, semo1)
        ibuf_b = (ibuf0, ibuf1)
        semt_b = (semt0, semt1)
        tcols = [jnp.full((_LANES,), t, jnp.int32) for t in range(4)]

        def iw_step(i, ip, first):
            win = jnp.minimum(i * _NUM_SUBCORES + sid, n_iwin - 1)
            r0 = win * _IW
            srcs = (a_hbm, b_hbm, g_hbm, m_hbm)
            for t, src in enumerate(srcs):
                pltpu.async_copy(src.at[pl.ds(r0, _IW)], stg_v.at[t], semi)
            @pl.when(jnp.logical_not(first))
            def _():
                pltpu.make_async_copy(
                    ibuf_b[ip], tab_hbm.at[cid, pl.ds(r0, _IW)],
                    semt_b[ip]).wait()
            for t, src in enumerate(srcs):
                pltpu.make_async_copy(
                    src.at[pl.ds(r0, _IW)], stg_v.at[t], semi).wait()

            def ivec(v, c2):
                o = v * _LANES
                rows = lanes + o
                for t in range(4):
                    plsc.store_scatter(ibuf_b[ip], [rows, tcols[t]],
                                       stg_v[t, pl.ds(o, _LANES)])
                return c2

            lax.fori_loop(0, _IW // _LANES, ivec, 0, unroll=4)
            pltpu.async_copy(ibuf_b[ip], tab_hbm.at[cid, pl.ds(r0, _IW)],
                             semt_b[ip])

        def iw_pair(j, carry):
            iw_step(2 * j, 0, j == 0)
            iw_step(2 * j + 1, 1, j == 0)
            return carry

        lax.fori_loop(0, n_my // 2, iw_pair, 0)
        for ip in range(2):
            pltpu.make_async_copy(
                ibuf_b[ip], tab_hbm.at[cid, pl.ds(0, _IW)],
                semt_b[ip]).wait()
        plsc.subcore_barrier()

        def launch(w, p):
            off = base + w * window
            pltpu.sync_copy(k_hbm.at[pl.ds(off, window)], idx_b[p])
            for j in range(n_grp):
                pltpu.async_copy(
                    tab_hbm.at[cid].at[idx_b[p].at[pl.ds(j * _GRP, _GRP)]],
                    buf_b[p].at[pl.ds(j * _GRP, _GRP)], sem_b[p])

        def out_descr(w, p, t):
            row0 = row_base + w * _WROWS
            return pltpu.make_async_copy(
                cols_b[p].at[t], out_hbm.at[t, pl.ds(row0, _WROWS)],
                semo_b[p])

        def finish(w, p, first):
            row0 = row_base + w * _WROWS
            for j in range(n_grp):
                pltpu.make_async_copy(
                    tab_hbm.at[cid].at[idx_b[p].at[pl.ds(j * _GRP, _GRP)]],
                    buf_b[p].at[pl.ds(j * _GRP, _GRP)], sem_b[p]).wait()

            @pl.when(jnp.logical_not(first))
            def _():
                for t in range(4):
                    out_descr(w, p, t).wait()

            def deint(r, c2):
                p0 = r * l
                for c0 in chunk0:
                    rows = lanes + (p0 + c0)
                    for t in range(4):
                        vec = plsc.load_gather(buf_b[p], [rows, tcols[t]])
                        cols_b[p][t, r, pl.ds(c0, _LANES)] = vec
                return c2

            lax.fori_loop(0, _WROWS, deint, 0)
            for t in range(4):
                pltpu.async_copy(
                    cols_b[p].at[t], out_hbm.at[t, pl.ds(row0, _WROWS)],
                    semo_b[p])

        launch(0, 0)

        def step(i, carry):
            w0 = 2 * i
            launch(w0 + 1, 1)
            finish(w0, 0, i == 0)

            @pl.when(i < n_win // 2 - 1)
            def _():
                launch(w0 + 2, 0)

            finish(w0 + 1, 1, i == 0)
            return carry

        lax.fori_loop(0, n_win // 2, step, 0)
        for p in range(2):
            for t in range(4):
                out_descr(n_win - 2 + p, p, t).wait()

    out, _ = body(kf, ak, bk, gk, mk)
    return out


def kernel(k, ak, bk, gk, mk):
    b, l = k.shape
    kf = k.reshape(b * l).astype(jnp.int32)
    return _gather4(kf, ak, bk, gk, mk, b=b, l=l)


# speedup vs baseline: 337.3548x; 1.0699x over previous
import functools

import jax
import jax.numpy as jnp
from jax import lax
from jax.experimental import pallas as pl
from jax.experimental.pallas import tpu as pltpu
from jax.experimental.pallas import tpu_sc as plsc

_NUM_CORES = 2
_NUM_SUBCORES = 16
_NUM_WORKERS = _NUM_CORES * _NUM_SUBCORES

_D = 8
_GRP = 128
_LANES = 16
_IW = 1600
_WROWS = 16


def _gather4(kf, ak, bk, gk, mk, *, b, l):
    n = kf.shape[0]
    v_rows = ak.shape[0]
    rows_per_w = b // _NUM_WORKERS
    window = _WROWS * l
    per_w = rows_per_w * l
    n_win = rows_per_w // _WROWS
    n_grp = window // _GRP
    assert rows_per_w % _WROWS == 0 and window % _GRP == 0 and n_win % 2 == 0
    n_iwin = v_rows // _IW
    assert v_rows % _IW == 0 and _IW % _LANES == 0
    n_my = -(-n_iwin // _NUM_SUBCORES)
    assert n_my % 2 == 0
    chunk0 = list(range(0, l - _LANES + 1, _LANES))
    if chunk0[-1] != l - _LANES:
        chunk0.append(l - _LANES)

    mesh = plsc.VectorSubcoreMesh(
        core_axis_name="c", subcore_axis_name="s",
        num_cores=_NUM_CORES, num_subcores=_NUM_SUBCORES)

    @functools.partial(
        pl.kernel,
        out_type=(
            jax.ShapeDtypeStruct((4, b, l), jnp.float32),
            jax.ShapeDtypeStruct((_NUM_CORES, v_rows, _D), jnp.float32),
        ),
        mesh=mesh,
        scratch_types=[
            pltpu.VMEM((window,), jnp.int32),
            pltpu.VMEM((window,), jnp.int32),
            pltpu.VMEM((window, _D), jnp.float32),
            pltpu.VMEM((window, _D), jnp.float32),
            pltpu.VMEM((4, _WROWS, l), jnp.float32),
            pltpu.VMEM((4, _WROWS, l), jnp.float32),
            pltpu.VMEM((4, _IW), jnp.float32),
            pltpu.VMEM((4, _IW), jnp.float32),
            pltpu.VMEM((_IW, _D), jnp.float32),
            pltpu.VMEM((_IW, _D), jnp.float32),
            pltpu.SemaphoreType.DMA,
            pltpu.SemaphoreType.DMA,
            pltpu.SemaphoreType.DMA,
            pltpu.SemaphoreType.DMA,
            pltpu.SemaphoreType.DMA,
            pltpu.SemaphoreType.DMA,
            pltpu.SemaphoreType.DMA,
            pltpu.SemaphoreType.DMA,
        ],
        compiler_params=pltpu.CompilerParams(
            use_tc_tiling_on_sc=False, needs_layout_passes=False),
    )
    def body(k_hbm, a_hbm, b_hbm, g_hbm, m_hbm, out_hbm, tab_hbm,
             idx0, idx1, buf0, buf1, cols0, cols1, stg0, stg1, ibuf0, ibuf1,
             sem0, sem1, semo0, semo1, semi0, semi1, semt0, semt1):
        cid = lax.axis_index("c")
        sid = lax.axis_index("s")
        wid = sid * _NUM_CORES + cid
        base = wid * per_w
        row_base = wid * rows_per_w
        lanes = lax.iota(jnp.int32, _LANES)
        idx_b = (idx0, idx1)
        buf_b = (buf0, buf1)
        cols_b = (cols0, cols1)
        sem_b = (sem0, sem1)
        semo_b = (semo0, semo1)
        ibuf_b = (ibuf0, ibuf1)
        semt_b = (semt0, semt1)
        stg_b = (stg0, stg1)
        semi_b = (semi0, semi1)
        tcols = [jnp.full((_LANES,), t, jnp.int32) for t in range(4)]
        srcs = (a_hbm, b_hbm, g_hbm, m_hbm)

        def iw_r0(i):
            return jnp.minimum(i * _NUM_SUBCORES + sid, n_iwin - 1) * _IW

        def iw_stage(i, sp):
            r0 = iw_r0(i)
            for t, src in enumerate(srcs):
                pltpu.async_copy(src.at[pl.ds(r0, _IW)], stg_b[sp].at[t],
                                 semi_b[sp])

        def iw_step(i, ip, first):
            r0 = iw_r0(i)
            for t, src in enumerate(srcs):
                pltpu.make_async_copy(
                    src.at[pl.ds(r0, _IW)], stg_b[ip].at[t],
                    semi_b[ip]).wait()

            @pl.when(i + 1 < n_my)
            def _():
                iw_stage(i + 1, 1 - ip)

            @pl.when(jnp.logical_not(first))
            def _():
                pltpu.make_async_copy(
                    ibuf_b[ip], tab_hbm.at[cid, pl.ds(r0, _IW)],
                    semt_b[ip]).wait()

            def ivec(v, c2):
                o = v * _LANES
                rows = lanes + o
                for t in range(4):
                    plsc.store_scatter(ibuf_b[ip], [rows, tcols[t]],
                                       stg_b[ip][t, pl.ds(o, _LANES)])
                return c2

            lax.fori_loop(0, _IW // _LANES, ivec, 0, unroll=4)
            pltpu.async_copy(ibuf_b[ip], tab_hbm.at[cid, pl.ds(r0, _IW)],
                             semt_b[ip])

        def iw_pair(j, carry):
            iw_step(2 * j, 0, j == 0)
            iw_step(2 * j + 1, 1, j == 0)
            return carry

        iw_stage(0, 0)
        lax.fori_loop(0, n_my // 2, iw_pair, 0)
        for ip in range(2):
            pltpu.make_async_copy(
                ibuf_b[ip], tab_hbm.at[cid, pl.ds(0, _IW)],
                semt_b[ip]).wait()
        plsc.subcore_barrier()

        def launch(w, p):
            off = base + w * window
            pltpu.sync_copy(k_hbm.at[pl.ds(off, window)], idx_b[p])
            for j in range(n_grp):
                pltpu.async_copy(
                    tab_hbm.at[cid].at[idx_b[p].at[pl.ds(j * _GRP, _GRP)]],
                    buf_b[p].at[pl.ds(j * _GRP, _GRP)], sem_b[p])

        def out_descr(w, p, t):
            row0 = row_base + w * _WROWS
            return pltpu.make_async_copy(
                cols_b[p].at[t], out_hbm.at[t, pl.ds(row0, _WROWS)],
                semo_b[p])

        def finish(w, p, first):
            row0 = row_base + w * _WROWS
            for j in range(n_grp):
                pltpu.make_async_copy(
                    tab_hbm.at[cid].at[idx_b[p].at[pl.ds(j * _GRP, _GRP)]],
                    buf_b[p].at[pl.ds(j * _GRP, _GRP)], sem_b[p]).wait()

            @pl.when(jnp.logical_not(first))
            def _():
                for t in range(4):
                    out_descr(w, p, t).wait()

            def deint(r, c2):
                p0 = r * l
                for c0 in chunk0:
                    rows = lanes + (p0 + c0)
                    for t in range(4):
                        vec = plsc.load_gather(buf_b[p], [rows, tcols[t]])
                        cols_b[p][t, r, pl.ds(c0, _LANES)] = vec
                return c2

            lax.fori_loop(0, _WROWS, deint, 0)
            for t in range(4):
                pltpu.async_copy(
                    cols_b[p].at[t], out_hbm.at[t, pl.ds(row0, _WROWS)],
                    semo_b[p])

        launch(0, 0)

        def step(i, carry):
            w0 = 2 * i
            launch(w0 + 1, 1)
            finish(w0, 0, i == 0)

            @pl.when(i < n_win // 2 - 1)
            def _():
                launch(w0 + 2, 0)

            finish(w0 + 1, 1, i == 0)
            return carry

        lax.fori_loop(0, n_win // 2, step, 0)
        for p in range(2):
            for t in range(4):
                out_descr(n_win - 2 + p, p, t).wait()

    out, _ = body(kf, ak, bk, gk, mk)
    return out


def kernel(k, ak, bk, gk, mk):
    b, l = k.shape
    kf = k.reshape(b * l).astype(jnp.int32)
    return _gather4(kf, ak, bk, gk, mk, b=b, l=l)
